# baseline (speedup 1.0000x reference)
import functools
import math

import jax
import jax.numpy as jnp
from jax import lax
from jax.experimental import pallas as pl
from jax.experimental.pallas import tpu as pltpu
from jax.experimental.pallas import tpu_sc as plsc

N = 10000
E = 320000
H = 128
EPS = 1e-5
NP = 10240
NC = 2
NS = 16
NW = NC * NS
RPT = NP // NS
EPW = E // NW
EC = 80
ICH = 25
SUP = ICH * EC
NSUP = EPW // SUP
AEC = 40
AICH = 5
ASUP = AEC * AICH
ANSUP = EPW // ASUP
NRB = 5
GA = 2
BL = 400
GRID = N // BL
_BN_SCALE = 1.0 / math.sqrt(1.0 + EPS)

_mesh = plsc.VectorSubcoreMesh(core_axis_name="c", subcore_axis_name="s")


@functools.partial(
    pl.kernel,
    out_type=jax.ShapeDtypeStruct((NC, NP), jnp.float32),
    mesh=_mesh,
    scratch_types=(
        [pltpu.VMEM((EC,), jnp.int32) for _ in range(ICH)]
        + [
            pltpu.VMEM((SUP,), jnp.float32),
            pltpu.VMEM((RPT,), jnp.float32),
            pltpu.SemaphoreType.DMA,
            pltpu.SemaphoreType.DMA,
            pltpu.VMEM_SHARED((NP,), jnp.float32),
        ]
    ),
)
def _sc_deg_kernel(c_hbm, w_hbm, deg_hbm, *scr):
    cbufs = scr[0:ICH]
    wv, degv, sem_st, sem_sc, deg_sh = scr[ICH:ICH + 5]
    cid = lax.axis_index("c")
    sid = lax.axis_index("s")
    base_n = sid * RPT

    def _zero(k, carry):
        degv[pl.ds(k * 16, 16)] = jnp.zeros((16,), jnp.float32)
        return carry
    lax.fori_loop(0, RPT // 16, _zero, 0)
    pltpu.sync_copy(degv, deg_sh.at[pl.ds(base_n, RPT)])
    plsc.subcore_barrier()

    wid = cid * NS + sid
    ebase = wid * EPW
    def _super(i, carry):
        off = ebase + i * SUP
        stages = [
            pltpu.async_copy(c_hbm.at[pl.ds(off + k * EC, EC)], cbufs[k],
                             sem_st)
            for k in range(ICH)
        ]
        stages.append(
            pltpu.async_copy(w_hbm.at[pl.ds(off, SUP)], wv, sem_st))
        for cp in stages:
            cp.wait()
        scatters = [
            pltpu.async_copy(wv.at[pl.ds(k * EC, EC)],
                             deg_sh.at[cbufs[k]], sem_sc, add=True)
            for k in range(ICH)
        ]
        for cp in scatters:
            cp.wait()
        return carry
    lax.fori_loop(0, NSUP, _super, 0)
    plsc.subcore_barrier()
    pltpu.sync_copy(deg_sh.at[pl.ds(base_n, RPT)],
                    deg_hbm.at[cid, pl.ds(base_n, RPT)])


@functools.partial(
    pl.kernel,
    out_type=jax.ShapeDtypeStruct((NC, NP, H), jnp.float32),
    mesh=_mesh,
    scratch_types=(
        [pltpu.VMEM((AEC,), jnp.int32) for _ in range(4 * AICH)]
        + [pltpu.VMEM((256,), jnp.float32) for _ in range(2)]
        + [pltpu.VMEM((AEC, H), jnp.float32) for _ in range(NRB)]
        + [pltpu.SemaphoreType.DMA for _ in range(2)]
        + [pltpu.SemaphoreType.DMA for _ in range(NRB)]
        + [pltpu.SemaphoreType.DMA for _ in range(NRB)]
        + [pltpu.VMEM_SHARED((NP, H), jnp.float32)]
    ),
)
def _sc_agg_kernel(act_hbm, r_hbm, c_hbm, w_hbm, out_hbm, *scr):
    _w0 = 4 * AICH
    _rb0 = _w0 + 2
    _sm0 = _rb0 + NRB
    setA = (scr[0:AICH], scr[2 * AICH:3 * AICH], scr[_w0], scr[_sm0])
    setB = (scr[AICH:2 * AICH], scr[3 * AICH:4 * AICH], scr[_w0 + 1],
            scr[_sm0 + 1])
    rowbufs = scr[_rb0:_rb0 + NRB]
    gsems = scr[_sm0 + 2:_sm0 + 2 + NRB]
    ssems = scr[_sm0 + 2 + NRB:_sm0 + 2 + 2 * NRB]
    acc_sh = scr[-1]
    cid = lax.axis_index("c")
    sid = lax.axis_index("s")
    base_n = sid * RPT

    rb0 = rowbufs[0]
    def _zrows(j, carry):
        for q in range(H // 16):
            rb0[j, pl.ds(q * 16, 16)] = jnp.zeros((16,), jnp.float32)
        return carry
    lax.fori_loop(0, AEC, _zrows, 0)
    zcps = [
        pltpu.async_copy(rb0, acc_sh.at[pl.ds(base_n + t * AEC, AEC)],
                         gsems[0])
        for t in range(RPT // AEC)
    ]
    for cp in zcps:
        cp.wait()
    plsc.subcore_barrier()

    wid = cid * NS + sid
    ebase = wid * EPW

    def stage_fire(off, st):
        rs, cs, wv, sem = st
        for k in range(AICH):
            pltpu.async_copy(r_hbm.at[pl.ds(off + k * AEC, AEC)], rs[k], sem)
            pltpu.async_copy(c_hbm.at[pl.ds(off + k * AEC, AEC)], cs[k], sem)
        pltpu.async_copy(w_hbm.at[pl.ds(off, ASUP)], wv.at[pl.ds(0, ASUP)],
                         sem)

    def stage_wait_r(off, st, j):
        rs, _, _, sem = st
        pltpu.make_async_copy(r_hbm.at[pl.ds(off + j * AEC, AEC)], rs[j],
                              sem).wait()

    def stage_wait_rest(off, st):
        rs, cs, wv, sem = st
        for k in range(GA, AICH):
            pltpu.make_async_copy(r_hbm.at[pl.ds(off + k * AEC, AEC)], rs[k],
                                  sem).wait()
        for k in range(AICH):
            pltpu.make_async_copy(c_hbm.at[pl.ds(off + k * AEC, AEC)], cs[k],
                                  sem).wait()
        pltpu.make_async_copy(w_hbm.at[pl.ds(off, ASUP)],
                              wv.at[pl.ds(0, ASUP)], sem).wait()

    def wait_scatter(b, st):
        pltpu.make_async_copy(rowbufs[b], acc_sh.at[st[1][b]],
                              ssems[b]).wait()

    def fire_gather(rref, b):
        pltpu.async_copy(act_hbm.at[rref], rowbufs[b], gsems[b])

    def wait_gather(rref, b):
        pltpu.make_async_copy(act_hbm.at[rref], rowbufs[b], gsems[b]).wait()

    def scale_chunk(rows, wv, k):
        base = k * AEC
        @plsc.parallel_loop(0, AEC, unroll=6)
        def _scale(j):
            j16 = (j >> 4) << 4
            nv = wv[pl.ds(pl.multiple_of(base + j16, 8), 16)]
            sv = lax.gather(
                nv, jnp.full((16, 1), j - j16, jnp.int32),
                lax.GatherDimensionNumbers(
                    offset_dims=(), collapsed_slice_dims=(0,),
                    start_index_map=(0,)),
                (1,), mode=lax.GatherScatterMode.PROMISE_IN_BOUNDS)
            for q in range(H // 16):
                rows[j, pl.ds(q * 16, 16)] = rows[j, pl.ds(q * 16, 16)] * sv

    def do_super(off, next_off, cur, nxt, first=False, last=False):
        if not first:
            stage_wait_rest(off, cur)
        for k in range(AICH):
            nk = k + GA
            if nk < AICH:
                if not first:
                    wait_scatter(nk, cur)
                fire_gather(cur[0][nk], nk)
            elif not last:
                j = nk - AICH
                stage_wait_r(next_off, nxt, j)
                wait_scatter(j, cur)
                fire_gather(nxt[0][j], j)
            if k == GA and not last:
                stage_fire(next_off, nxt)
            wait_gather(cur[0][k], k)
            scale_chunk(rowbufs[k], cur[2], k)
            pltpu.async_copy(rowbufs[k], acc_sh.at[cur[1][k]], ssems[k],
                             add=True)

    stage_fire(ebase, setA)
    stage_wait_r(ebase, setA, 0)
    stage_wait_r(ebase, setA, 1)
    stage_wait_rest(ebase, setA)
    for j in range(GA):
        fire_gather(setA[0][j], j)
    do_super(ebase, ebase + ASUP, setA, setB, first=True)

    def _pair(p, carry):
        off1 = ebase + (2 * p + 1) * ASUP
        do_super(off1, off1 + ASUP, setB, setA)
        do_super(off1 + ASUP, off1 + 2 * ASUP, setA, setB)
        return carry
    lax.fori_loop(0, (ANSUP - 2) // 2, _pair, 0)
    off_last = ebase + (ANSUP - 1) * ASUP
    do_super(off_last, off_last, setB, setA, last=True)

    for b in range(NRB):
        wait_scatter(b, setB)
    plsc.subcore_barrier()
    pltpu.sync_copy(acc_sh.at[pl.ds(base_n, RPT)],
                    out_hbm.at[cid, pl.ds(base_n, RPT)])


def _tc_pre_body(x_ref, deg_ref, w_ref, b_ref, g_ref, be_ref, o_ref):
    dinv = lax.rsqrt(deg_ref[0] + deg_ref[1] + 2.0)
    z = lax.dot_general(x_ref[...], w_ref[...], (((1,), (1,)), ((), ())),
                        preferred_element_type=jnp.float32)
    sc = g_ref[...] * _BN_SCALE
    a = jnp.maximum(z * sc + (b_ref[...] * sc + be_ref[...]), 0.0)
    o_ref[...] = dinv * a


_tc_pre = pl.pallas_call(
    _tc_pre_body,
    grid=(GRID,),
    in_specs=[
        pl.BlockSpec((BL, H), lambda i: (i, 0)),
        pl.BlockSpec((NC, BL, 1), lambda i: (0, i, 0)),
        pl.BlockSpec((H, H), lambda i: (0, 0)),
        pl.BlockSpec((1, H), lambda i: (0, 0)),
        pl.BlockSpec((1, H), lambda i: (0, 0)),
        pl.BlockSpec((1, H), lambda i: (0, 0)),
    ],
    out_specs=pl.BlockSpec((BL, H), lambda i: (i, 0)),
    out_shape=jax.ShapeDtypeStruct((N, H), jnp.float32),
)


def _tc_layer_mid_body(agg_ref, dact_ref, deg_ref, w_ref, b_ref, g_ref,
                       be_ref, dact_ref_o):
    dinv = lax.rsqrt(deg_ref[0] + deg_ref[1] + 2.0)
    m = dinv * (agg_ref[0] + agg_ref[1] + 2.0 * dact_ref[...])
    z = lax.dot_general(m, w_ref[...], (((1,), (1,)), ((), ())),
                        preferred_element_type=jnp.float32)
    sc = g_ref[...] * _BN_SCALE
    a = jnp.maximum(z * sc + (b_ref[...] * sc + be_ref[...]), 0.0)
    dact_ref_o[...] = dinv * a


def _tc_layer_fin_body(agg_ref, dact_ref, deg_ref, w_ref, b_ref, g_ref,
                       be_ref, act_ref_o):
    dinv = lax.rsqrt(deg_ref[0] + deg_ref[1] + 2.0)
    m = dinv * (agg_ref[0] + agg_ref[1] + 2.0 * dact_ref[...])
    z = lax.dot_general(m, w_ref[...], (((1,), (1,)), ((), ())),
                        preferred_element_type=jnp.float32)
    sc = g_ref[...] * _BN_SCALE
    act_ref_o[...] = jnp.maximum(z * sc + (b_ref[...] * sc + be_ref[...]),
                                 0.0)


_TC_LAYER_SPECS = dict(
    grid=(GRID,),
    in_specs=[
        pl.BlockSpec((NC, BL, H), lambda i: (0, i, 0)),
        pl.BlockSpec((BL, H), lambda i: (i, 0)),
        pl.BlockSpec((NC, BL, 1), lambda i: (0, i, 0)),
        pl.BlockSpec((H, H), lambda i: (0, 0)),
        pl.BlockSpec((1, H), lambda i: (0, 0)),
        pl.BlockSpec((1, H), lambda i: (0, 0)),
        pl.BlockSpec((1, H), lambda i: (0, 0)),
    ],
    out_specs=pl.BlockSpec((BL, H), lambda i: (i, 0)),
    out_shape=jax.ShapeDtypeStruct((N, H), jnp.float32),
)

_tc_layer_mid = pl.pallas_call(_tc_layer_mid_body, **_TC_LAYER_SPECS)
_tc_layer_fin = pl.pallas_call(_tc_layer_fin_body, **_TC_LAYER_SPECS)


def kernel(x, edge_index, edge_weight, batch, W_pre, b_pre, g_pre, be_pre,
           Wc0, bc0, gc0, bec0, Wc1, bc1, gc1, bec1, Wc2, bc2, gc2, bec2):
    del batch
    r = edge_index[0]
    c = edge_index[1]

    deg = _sc_deg_kernel(c, edge_weight)
    deg3 = deg.reshape(NC, NP, 1)[:, :N]
    dact = _tc_pre(x, deg3, W_pre, b_pre.reshape(1, H), g_pre.reshape(1, H),
                   be_pre.reshape(1, H))
    for li, (W, b, g, be) in enumerate(
            ((Wc0, bc0, gc0, bec0), (Wc1, bc1, gc1, bec1),
             (Wc2, bc2, gc2, bec2))):
        agg = _sc_agg_kernel(dact, r, c, edge_weight)
        fn = _tc_layer_fin if li == 2 else _tc_layer_mid
        dact = fn(agg, dact, deg3, W, b.reshape(1, H), g.reshape(1, H),
                  be.reshape(1, H))
    return dact


# speedup vs baseline: 1.0845x; 1.0845x over previous
import functools
import math

import jax
import jax.numpy as jnp
from jax import lax
from jax.experimental import pallas as pl
from jax.experimental.pallas import tpu as pltpu
from jax.experimental.pallas import tpu_sc as plsc

N = 10000
E = 320000
H = 128
EPS = 1e-5
NP = 10240
NC = 2
NS = 16
NW = NC * NS
RPT = NP // NS
EPW = E // NW
EC = 80
ICH = 25
SUP = ICH * EC
NSUP = EPW // SUP
AEC = 40
AICH = 5
ASUP = AEC * AICH
ANSUP = EPW // ASUP
NRB = 5
GA = 2
BL = 400
GRID = N // BL
_BN_SCALE = 1.0 / math.sqrt(1.0 + EPS)

_mesh = plsc.VectorSubcoreMesh(core_axis_name="c", subcore_axis_name="s")


@functools.partial(
    pl.kernel,
    out_type=jax.ShapeDtypeStruct((NC, NP), jnp.float32),
    mesh=_mesh,
    scratch_types=(
        [pltpu.VMEM((EC,), jnp.int32) for _ in range(ICH)]
        + [
            pltpu.VMEM((SUP,), jnp.float32),
            pltpu.VMEM((RPT,), jnp.float32),
            pltpu.SemaphoreType.DMA,
            pltpu.SemaphoreType.DMA,
            pltpu.VMEM_SHARED((NP,), jnp.float32),
        ]
    ),
)
def _sc_deg_kernel(c_hbm, w_hbm, deg_hbm, *scr):
    cbufs = scr[0:ICH]
    wv, degv, sem_st, sem_sc, deg_sh = scr[ICH:ICH + 5]
    cid = lax.axis_index("c")
    sid = lax.axis_index("s")
    base_n = sid * RPT

    def _zero(k, carry):
        degv[pl.ds(k * 16, 16)] = jnp.zeros((16,), jnp.float32)
        return carry
    lax.fori_loop(0, RPT // 16, _zero, 0)
    pltpu.sync_copy(degv, deg_sh.at[pl.ds(base_n, RPT)])
    plsc.subcore_barrier()

    wid = cid * NS + sid
    ebase = wid * EPW
    def _super(i, carry):
        off = ebase + i * SUP
        stages = [
            pltpu.async_copy(c_hbm.at[pl.ds(off + k * EC, EC)], cbufs[k],
                             sem_st)
            for k in range(ICH)
        ]
        stages.append(
            pltpu.async_copy(w_hbm.at[pl.ds(off, SUP)], wv, sem_st))
        for cp in stages:
            cp.wait()
        scatters = [
            pltpu.async_copy(wv.at[pl.ds(k * EC, EC)],
                             deg_sh.at[cbufs[k]], sem_sc, add=True)
            for k in range(ICH)
        ]
        for cp in scatters:
            cp.wait()
        return carry
    lax.fori_loop(0, NSUP, _super, 0)
    plsc.subcore_barrier()
    pltpu.sync_copy(deg_sh.at[pl.ds(base_n, RPT)],
                    deg_hbm.at[cid, pl.ds(base_n, RPT)])


@functools.partial(
    pl.kernel,
    out_type=jax.ShapeDtypeStruct((NC, NP, H), jnp.float32),
    mesh=_mesh,
    scratch_types=(
        [pltpu.VMEM((AEC,), jnp.int32) for _ in range(4 * AICH)]
        + [pltpu.VMEM((256,), jnp.float32) for _ in range(2)]
        + [pltpu.VMEM((AEC, H), jnp.float32) for _ in range(NRB)]
        + [pltpu.SemaphoreType.DMA for _ in range(2)]
        + [pltpu.SemaphoreType.DMA for _ in range(NRB)]
        + [pltpu.SemaphoreType.DMA for _ in range(NRB)]
        + [pltpu.VMEM_SHARED((NP, H), jnp.float32)]
    ),
)
def _sc_agg_kernel(act_hbm, r_hbm, c_hbm, w_hbm, out_hbm, *scr):
    _w0 = 4 * AICH
    _rb0 = _w0 + 2
    _sm0 = _rb0 + NRB
    setA = (scr[0:AICH], scr[2 * AICH:3 * AICH], scr[_w0], scr[_sm0])
    setB = (scr[AICH:2 * AICH], scr[3 * AICH:4 * AICH], scr[_w0 + 1],
            scr[_sm0 + 1])
    rowbufs = scr[_rb0:_rb0 + NRB]
    gsems = scr[_sm0 + 2:_sm0 + 2 + NRB]
    ssems = scr[_sm0 + 2 + NRB:_sm0 + 2 + 2 * NRB]
    acc_sh = scr[-1]
    cid = lax.axis_index("c")
    sid = lax.axis_index("s")
    base_n = sid * RPT

    rb0 = rowbufs[0]
    def _zrows(j, carry):
        for q in range(H // 16):
            rb0[j, pl.ds(q * 16, 16)] = jnp.zeros((16,), jnp.float32)
        return carry
    lax.fori_loop(0, AEC, _zrows, 0)
    zcps = [
        pltpu.async_copy(rb0, acc_sh.at[pl.ds(base_n + t * AEC, AEC)],
                         gsems[0])
        for t in range(RPT // AEC)
    ]
    for cp in zcps:
        cp.wait()
    plsc.subcore_barrier()

    wid = cid * NS + sid
    ebase = wid * EPW

    def stage_fire(off, st):
        rs, cs, wv, sem = st
        for k in range(AICH):
            pltpu.async_copy(r_hbm.at[pl.ds(off + k * AEC, AEC)], rs[k], sem)
            pltpu.async_copy(c_hbm.at[pl.ds(off + k * AEC, AEC)], cs[k], sem)
        pltpu.async_copy(w_hbm.at[pl.ds(off, ASUP)], wv.at[pl.ds(0, ASUP)],
                         sem)

    def stage_wait_r(off, st, j):
        rs, _, _, sem = st
        pltpu.make_async_copy(r_hbm.at[pl.ds(off + j * AEC, AEC)], rs[j],
                              sem).wait()

    def stage_wait_rest(off, st):
        rs, cs, wv, sem = st
        for k in range(GA, AICH):
            pltpu.make_async_copy(r_hbm.at[pl.ds(off + k * AEC, AEC)], rs[k],
                                  sem).wait()
        for k in range(AICH):
            pltpu.make_async_copy(c_hbm.at[pl.ds(off + k * AEC, AEC)], cs[k],
                                  sem).wait()
        pltpu.make_async_copy(w_hbm.at[pl.ds(off, ASUP)],
                              wv.at[pl.ds(0, ASUP)], sem).wait()

    def wait_scatter(b, st):
        pltpu.make_async_copy(rowbufs[b], acc_sh.at[st[1][b]],
                              ssems[b]).wait()

    def fire_gather(rref, b):
        pltpu.async_copy(act_hbm.at[rref], rowbufs[b], gsems[b])

    def wait_gather(rref, b):
        pltpu.make_async_copy(act_hbm.at[rref], rowbufs[b], gsems[b]).wait()

    def scale_chunk(rows, wv, k):
        base = k * AEC
        @plsc.parallel_loop(0, AEC, unroll=4)
        def _scale(j):
            j16 = (j >> 4) << 4
            nv = wv[pl.ds(pl.multiple_of(base + j16, 8), 16)]
            sv = lax.gather(
                nv, jnp.full((16, 1), j - j16, jnp.int32),
                lax.GatherDimensionNumbers(
                    offset_dims=(), collapsed_slice_dims=(0,),
                    start_index_map=(0,)),
                (1,), mode=lax.GatherScatterMode.PROMISE_IN_BOUNDS)
            for q in range(H // 16):
                rows[j, pl.ds(q * 16, 16)] = rows[j, pl.ds(q * 16, 16)] * sv

    def do_super(off, next_off, cur, nxt, first=False, last=False):
        if not first:
            stage_wait_rest(off, cur)
        for k in range(AICH):
            nk = k + GA
            if nk < AICH:
                if not first:
                    wait_scatter(nk, cur)
                fire_gather(cur[0][nk], nk)
            elif not last:
                j = nk - AICH
                stage_wait_r(next_off, nxt, j)
                wait_scatter(j, cur)
                fire_gather(nxt[0][j], j)
            if k == GA and not last:
                stage_fire(next_off, nxt)
            wait_gather(cur[0][k], k)
            scale_chunk(rowbufs[k], cur[2], k)
            pltpu.async_copy(rowbufs[k], acc_sh.at[cur[1][k]], ssems[k],
                             add=True)

    stage_fire(ebase, setA)
    stage_wait_r(ebase, setA, 0)
    stage_wait_r(ebase, setA, 1)
    stage_wait_rest(ebase, setA)
    for j in range(GA):
        fire_gather(setA[0][j], j)
    do_super(ebase, ebase + ASUP, setA, setB, first=True)

    def _pair(p, carry):
        off1 = ebase + (2 * p + 1) * ASUP
        do_super(off1, off1 + ASUP, setB, setA)
        do_super(off1 + ASUP, off1 + 2 * ASUP, setA, setB)
        return carry
    lax.fori_loop(0, (ANSUP - 2) // 2, _pair, 0)
    off_last = ebase + (ANSUP - 1) * ASUP
    do_super(off_last, off_last, setB, setA, last=True)

    for b in range(NRB):
        wait_scatter(b, setB)
    plsc.subcore_barrier()
    pltpu.sync_copy(acc_sh.at[pl.ds(base_n, RPT)],
                    out_hbm.at[cid, pl.ds(base_n, RPT)])


def _tc_pre_body(x_ref, deg_ref, w_ref, b_ref, g_ref, be_ref, o_ref):
    dinv = lax.rsqrt(deg_ref[0] + deg_ref[1] + 2.0)
    z = lax.dot_general(x_ref[...], w_ref[...], (((1,), (1,)), ((), ())),
                        preferred_element_type=jnp.float32)
    sc = g_ref[...] * _BN_SCALE
    a = jnp.maximum(z * sc + (b_ref[...] * sc + be_ref[...]), 0.0)
    o_ref[...] = dinv * a


_tc_pre = pl.pallas_call(
    _tc_pre_body,
    grid=(GRID,),
    in_specs=[
        pl.BlockSpec((BL, H), lambda i: (i, 0)),
        pl.BlockSpec((NC, BL, 1), lambda i: (0, i, 0)),
        pl.BlockSpec((H, H), lambda i: (0, 0)),
        pl.BlockSpec((1, H), lambda i: (0, 0)),
        pl.BlockSpec((1, H), lambda i: (0, 0)),
        pl.BlockSpec((1, H), lambda i: (0, 0)),
    ],
    out_specs=pl.BlockSpec((BL, H), lambda i: (i, 0)),
    out_shape=jax.ShapeDtypeStruct((N, H), jnp.float32),
)


def _tc_layer_mid_body(agg_ref, dact_ref, deg_ref, w_ref, b_ref, g_ref,
                       be_ref, dact_ref_o):
    dinv = lax.rsqrt(deg_ref[0] + deg_ref[1] + 2.0)
    m = dinv * (agg_ref[0] + agg_ref[1] + 2.0 * dact_ref[...])
    z = lax.dot_general(m, w_ref[...], (((1,), (1,)), ((), ())),
                        preferred_element_type=jnp.float32)
    sc = g_ref[...] * _BN_SCALE
    a = jnp.maximum(z * sc + (b_ref[...] * sc + be_ref[...]), 0.0)
    dact_ref_o[...] = dinv * a


def _tc_layer_fin_body(agg_ref, dact_ref, deg_ref, w_ref, b_ref, g_ref,
                       be_ref, act_ref_o):
    dinv = lax.rsqrt(deg_ref[0] + deg_ref[1] + 2.0)
    m = dinv * (agg_ref[0] + agg_ref[1] + 2.0 * dact_ref[...])
    z = lax.dot_general(m, w_ref[...], (((1,), (1,)), ((), ())),
                        preferred_element_type=jnp.float32)
    sc = g_ref[...] * _BN_SCALE
    act_ref_o[...] = jnp.maximum(z * sc + (b_ref[...] * sc + be_ref[...]),
                                 0.0)


_TC_LAYER_SPECS = dict(
    grid=(GRID,),
    in_specs=[
        pl.BlockSpec((NC, BL, H), lambda i: (0, i, 0)),
        pl.BlockSpec((BL, H), lambda i: (i, 0)),
        pl.BlockSpec((NC, BL, 1), lambda i: (0, i, 0)),
        pl.BlockSpec((H, H), lambda i: (0, 0)),
        pl.BlockSpec((1, H), lambda i: (0, 0)),
        pl.BlockSpec((1, H), lambda i: (0, 0)),
        pl.BlockSpec((1, H), lambda i: (0, 0)),
    ],
    out_specs=pl.BlockSpec((BL, H), lambda i: (i, 0)),
    out_shape=jax.ShapeDtypeStruct((N, H), jnp.float32),
)

_tc_layer_mid = pl.pallas_call(_tc_layer_mid_body, **_TC_LAYER_SPECS)
_tc_layer_fin = pl.pallas_call(_tc_layer_fin_body, **_TC_LAYER_SPECS)


def kernel(x, edge_index, edge_weight, batch, W_pre, b_pre, g_pre, be_pre,
           Wc0, bc0, gc0, bec0, Wc1, bc1, gc1, bec1, Wc2, bc2, gc2, bec2):
    del batch
    r = edge_index[0]
    c = edge_index[1]

    deg = _sc_deg_kernel(c, edge_weight)
    deg3 = deg.reshape(NC, NP, 1)[:, :N]
    dact = _tc_pre(x, deg3, W_pre, b_pre.reshape(1, H), g_pre.reshape(1, H),
                   be_pre.reshape(1, H))
    for li, (W, b, g, be) in enumerate(
            ((Wc0, bc0, gc0, bec0), (Wc1, bc1, gc1, bec1),
             (Wc2, bc2, gc2, bec2))):
        agg = _sc_agg_kernel(dact, r, c, edge_weight)
        fn = _tc_layer_fin if li == 2 else _tc_layer_mid
        dact = fn(agg, dact, deg3, W, b.reshape(1, H), g.reshape(1, H),
                  be.reshape(1, H))
    return dact
